# baseline (speedup 1.0000x reference)
import functools

import jax
import jax.numpy as jnp
from jax import lax
from jax.experimental import pallas as pl
from jax.experimental.pallas import tpu as pltpu
from jax.experimental.pallas import tpu_sc as plsc

B, K, C = 4, 256, 128
L = 65536
RADIUS = 1.5
MAXN = 64
NCELL = 64
NGRP = B * NCELL
SCALE = 1.0 / (C ** 0.5)


_CODES_BLK = 8192


def _codes_body(spif_ref, codes_ref):
    blk = spif_ref[...]
    codes_ref[...] = (blk[:, 0] * 64 + blk[:, 1] * 16 + blk[:, 2] * 4
                      + blk[:, 3])


def _compute_codes(spif):
    return pl.pallas_call(
        _codes_body,
        grid=(L // _CODES_BLK,),
        in_specs=[pl.BlockSpec((_CODES_BLK, 4), lambda i: (i, 0))],
        out_specs=pl.BlockSpec((_CODES_BLK,), lambda i: (i,)),
        out_shape=jax.ShapeDtypeStruct((L,), jnp.int32),
    )(spif)


def _rank_body(qx_ref, dist_s_ref, cell_s_ref):
    qx = qx_ref[...]
    ci = lax.broadcasted_iota(jnp.int32, (1, NCELL), 1)
    px = (ci % 4).astype(jnp.float32) - 1.5
    py = ((ci // 4) % 4).astype(jnp.float32) - 1.5
    pz = (ci // 16).astype(jnp.float32) - 1.5
    qx0, qx1, qx2 = qx[:, 0:1], qx[:, 1:2], qx[:, 2:3]
    qb0 = qx0.astype(jnp.bfloat16).astype(jnp.float32)
    qb1 = qx1.astype(jnp.bfloat16).astype(jnp.float32)
    qb2 = qx2.astype(jnp.bfloat16).astype(jnp.float32)
    qs = (qx0 * qx0 + qx2 * qx2) + qx1 * qx1
    ps = (px * px + pz * pz) + py * py
    dot = qb0 * px + qb1 * py + qb2 * pz
    d2 = (qs + ps) - 2.0 * dot
    dist = jnp.sqrt(jnp.maximum(d2, 0.0))

    ci64 = lax.broadcasted_iota(jnp.int32, (B * K, NCELL), 1)
    rank = jnp.zeros((B * K, NCELL), jnp.int32)
    for cp in range(NCELL):
        dcp = dist[:, cp:cp + 1]
        rank = rank + (dcp < dist).astype(jnp.int32) \
            + jnp.where((dcp == dist) & (cp < ci64), 1, 0)
    dist_s = jnp.zeros_like(dist)
    cell_s = jnp.zeros((B * K, NCELL), jnp.int32)
    for c0 in range(NCELL):
        m = rank[:, c0:c0 + 1] == ci64
        dist_s = jnp.where(m, dist[:, c0:c0 + 1], dist_s)
        cell_s = jnp.where(m, c0, cell_s)
    dist_s_ref[...] = dist_s
    cell_s_ref[...] = cell_s


def _rank_cells(qx_flat):
    return pl.pallas_call(
        _rank_body,
        out_shape=(jax.ShapeDtypeStruct((B * K, NCELL), jnp.float32),
                   jax.ShapeDtypeStruct((B * K, NCELL), jnp.int32)),
    )(qx_flat)


_SC_CHUNK = 4096


@functools.cache
def _get_sc_count():
    mesh = plsc.VectorSubcoreMesh(core_axis_name="c", subcore_axis_name="s")
    return pl.kernel(
        _sc_count_body,
        out_type=(jax.ShapeDtypeStruct((NGRP * MAXN,), jnp.int32),
                  jax.ShapeDtypeStruct((NGRP,), jnp.int32)),
        mesh=mesh,
        scratch_types=[
            pltpu.VMEM((_SC_CHUNK,), jnp.int32),
            pltpu.VMEM((NGRP * MAXN,), jnp.int32),
            pltpu.VMEM((NGRP,), jnp.int32),
        ],
        compiler_params=pltpu.CompilerParams(needs_layout_passes=False),
    )


def _sc_count_body(codes_hbm, t_hbm, counts_hbm, codes_v, t_v, cnt_v):
    wid = lax.axis_index("s") * 2 + lax.axis_index("c")

    @pl.when(wid == 0)
    def _():
        zeros16 = jnp.zeros((16,), jnp.int32)
        for i in range(NGRP // 16):
            cnt_v[pl.ds(i * 16, 16)] = zeros16

        def zero_t(i, _):
            t_v[pl.ds(i * 16, 16)] = zeros16
            return _
        lax.fori_loop(0, NGRP * MAXN // 16, zero_t, 0)

        iota16 = lax.iota(jnp.int32, 16)
        zc, _ = plsc.scan_count(zeros16)
        basev = zc - iota16

        def chunk_body(ch, _):
            pltpu.sync_copy(codes_hbm.at[pl.ds(ch * _SC_CHUNK, _SC_CHUNK)],
                            codes_v)

            def step(j, _):
                g = codes_v[pl.ds(j * 16, 16)]
                cn = plsc.load_gather(cnt_v, [g])
                dup, lastm = plsc.scan_count(g)
                pos = cn + (dup - basev)
                wm = pos < MAXN
                idxv = ch * _SC_CHUNK + j * 16 + iota16
                plsc.store_scatter(t_v, [g * MAXN + pos], idxv, mask=wm)
                plsc.store_scatter(cnt_v, [g], pos + 1, mask=lastm)
                return _
            lax.fori_loop(0, _SC_CHUNK // 16, step, 0)
            return _
        lax.fori_loop(0, L // _SC_CHUNK, chunk_body, 0)

        pltpu.sync_copy(t_v, t_hbm)
        pltpu.sync_copy(cnt_v, counts_hbm)


_QB = 64
_JMAX = 8


def _walk(cnt, cell_s, t64):
    q = cnt.shape[0]
    s_inc = cnt
    for sh in (1, 2, 4, 8, 16, 32):
        s_inc = s_inc + jnp.concatenate(
            [jnp.zeros((q, sh), jnp.int32), s_inc[:, :NCELL - sh]], axis=1)
    s_exc = s_inc - cnt
    r_t = jnp.zeros((q, NCELL), jnp.int32)
    for r0 in range(NCELL):
        r_t = r_t + (s_inc[:, r0:r0 + 1] <= t64).astype(jnp.int32)
    rc = jnp.minimum(r_t, NCELL - 1)
    cellt = jnp.zeros((q, NCELL), jnp.int32)
    sexg = jnp.zeros((q, NCELL), jnp.int32)
    for r0 in range(NCELL):
        msk = rc == r0
        cellt = jnp.where(msk, cell_s[:, r0:r0 + 1], cellt)
        sexg = jnp.where(msk, s_exc[:, r0:r0 + 1], sexg)
    o_i = jnp.clip(t64 - sexg, 0, MAXN - 1)
    return s_inc, s_exc, r_t, rc, cellt, o_i


def _slots_body(dist_s_ref, cell_s_ref, counts_ref, t_ref,
                tidx_ref, cellt_ref, valid_ref):
    b = pl.program_id(0) // (B * K // _QB // B)
    dist_s = dist_s_ref[0]
    cell_s = cell_s_ref[0]
    cnt_s = jnp.zeros((_QB, NCELL), jnp.int32)
    for c0 in range(NCELL):
        cnt_s = jnp.where(cell_s == c0, counts_ref[0, 0, c0], cnt_s)
    t64 = lax.broadcasted_iota(jnp.int32, (_QB, NCELL), 1)
    s_inc, s_exc, r_t, rc, cellt, o_i = _walk(cnt_s, cell_s, t64)
    distg = jnp.zeros((_QB, NCELL), jnp.float32)
    for r0 in range(NCELL):
        distg = jnp.where(rc == r0, dist_s[:, r0:r0 + 1], distg)
    ok = r_t < NCELL
    valid = ok & (distg <= RADIUS)
    tidx_ref[0] = (b * NCELL + cellt) * MAXN + o_i
    cellt_ref[0] = cellt
    valid_ref[0] = valid.astype(jnp.float32)

    neq = jnp.concatenate(
        [jnp.zeros((_QB, 1), jnp.int32),
         (dist_s[:, 1:] != dist_s[:, :-1]).astype(jnp.int32)], axis=1)
    runid = neq
    for sh in (1, 2, 4, 8, 16, 32):
        runid = runid + jnp.concatenate(
            [jnp.zeros((_QB, sh), jnp.int32), runid[:, :NCELL - sh]], axis=1)
    r_star = r_t[:, NCELL - 1:NCELL]
    rc_star = jnp.minimum(r_star, NCELL - 1)
    run_star = jnp.zeros((_QB, 1), jnp.int32)
    r_lo = jnp.zeros((_QB, 1), jnp.int32)
    r_hi = jnp.zeros((_QB, 1), jnp.int32)
    rexc = jnp.zeros((_QB, 1), jnp.int32)
    rinc = jnp.zeros((_QB, 1), jnp.int32)
    for r0 in range(NCELL):
        run_star = jnp.where(rc_star == r0, runid[:, r0:r0 + 1], run_star)
    for r0 in range(NCELL):
        r_lo = r_lo + (runid[:, r0:r0 + 1] < run_star).astype(jnp.int32)
        r_hi = r_hi + (runid[:, r0:r0 + 1] <= run_star).astype(jnp.int32)
    r_hi = r_hi - 1
    for r0 in range(NCELL):
        rexc = jnp.where(r_lo == r0, s_exc[:, r0:r0 + 1], rexc)
        rinc = jnp.where(r_hi == r0, s_inc[:, r0:r0 + 1], rinc)
    jrun = r_hi - r_lo + 1
    mneed = MAXN - rexc
    dist_star = distg[:, NCELL - 1:NCELL]
    flag = ((r_star < NCELL) & (jrun >= 2) & (jrun <= _JMAX)
            & (rinc > MAXN) & (dist_star <= RADIUS))
    anyflag = jnp.max(flag.astype(jnp.int32)) > 0

    @pl.when(anyflag)
    def _fix():
        cap_s = jnp.minimum(cnt_s, MAXN)
        iota_r = lax.broadcasted_iota(jnp.int32, (_QB, NCELL), 1)
        cellrun = jnp.zeros((_QB, _JMAX), jnp.int32)
        caprun = jnp.zeros((_QB, _JMAX), jnp.int32)
        iota_jk = lax.broadcasted_iota(jnp.int32, (_QB, _JMAX), 1)
        for k in range(_JMAX):
            msk = (iota_r == r_lo + k).astype(jnp.int32)
            cellrun = jnp.where(iota_jk == k,
                                jnp.sum(msk * cell_s, axis=1, keepdims=True),
                                cellrun)
            caprun = jnp.where(iota_jk == k,
                               jnp.sum(msk * cap_s, axis=1, keepdims=True),
                               caprun)
        ncol = _JMAX * MAXN
        col = lax.broadcasted_iota(jnp.int32, (_QB, ncol), 1)
        k_of = col // MAXN
        t_of = col % MAXN
        cellrun_w = jnp.zeros((_QB, ncol), jnp.int32)
        caprun_w = jnp.zeros((_QB, ncol), jnp.int32)
        for k in range(_JMAX):
            km = k_of == k
            cellrun_w = jnp.where(km, cellrun[:, k:k + 1], cellrun_w)
            caprun_w = jnp.where(km, caprun[:, k:k + 1], caprun_w)
        t3 = jnp.zeros((_QB, ncol), jnp.float32)
        for c0 in range(NCELL):
            trow = t_ref[0, c0].astype(jnp.float32)
            trow_w = jnp.concatenate([trow] * _JMAX, axis=0)[None, :]
            t3 = t3 + (cellrun_w == c0).astype(jnp.float32) * trow_w
        emask = (k_of < jrun) & (t_of < caprun_w)
        mf = mneed.astype(jnp.float32)
        v = jnp.zeros((_QB, 1), jnp.float32)
        for bit in range(15, -1, -1):
            cand = v + float(1 << bit)
            cnt_lt = jnp.sum(jnp.where(emask & (t3 < cand), 1.0, 0.0),
                             axis=1, keepdims=True)
            v = jnp.where(cnt_lt < mf, cand, v)
        sel = jnp.where(emask & (t3 < v + 1.0), 1.0, 0.0)
        cnt2 = cnt_s
        for k in range(_JMAX):
            pk = jnp.sum(jnp.where(k_of == k, sel, 0.0), axis=1,
                         keepdims=True).astype(jnp.int32)
            mskr = (iota_r == r_lo + k) & flag & (k < jrun)
            cnt2 = jnp.where(mskr, pk, cnt2)
        _, _, _, _, cellt2, o2 = _walk(cnt2, cell_s, t64)
        flagb = flag
        tidx_ref[0] = jnp.where(flagb, (b * NCELL + cellt2) * MAXN + o2,
                                tidx_ref[0])
        cellt_ref[0] = jnp.where(flagb, cellt2, cellt_ref[0])


def _slot_walk(dist_s3, cell_s3, counts4, t4):
    nblk = B * K // _QB
    per_b = nblk // B
    return pl.pallas_call(
        _slots_body,
        grid=(nblk,),
        in_specs=[
            pl.BlockSpec((1, _QB, NCELL), lambda i: (i, 0, 0)),
            pl.BlockSpec((1, _QB, NCELL), lambda i: (i, 0, 0)),
            pl.BlockSpec((1, 1, NCELL), lambda i: (i // per_b, 0, 0)),
            pl.BlockSpec((1, NCELL, MAXN), lambda i: (i // per_b, 0, 0)),
        ],
        out_specs=(
            pl.BlockSpec((1, _QB, MAXN), lambda i: (i, 0, 0)),
            pl.BlockSpec((1, _QB, MAXN), lambda i: (i, 0, 0)),
            pl.BlockSpec((1, _QB, MAXN), lambda i: (i, 0, 0)),
        ),
        out_shape=(
            jax.ShapeDtypeStruct((nblk, _QB, MAXN), jnp.int32),
            jax.ShapeDtypeStruct((nblk, _QB, MAXN), jnp.int32),
            jax.ShapeDtypeStruct((nblk, _QB, MAXN), jnp.float32),
        ),
    )(dist_s3, cell_s3, counts4, t4)


_ROWS_PER_W = (B * K * MAXN) // 32
_GROW = 128


@functools.cache
def _get_sc_gather():
    mesh = plsc.VectorSubcoreMesh(core_axis_name="c", subcore_axis_name="s")
    return pl.kernel(
        _sc_gather_body,
        out_type=jax.ShapeDtypeStruct((B * K * MAXN, C), jnp.float32),
        mesh=mesh,
        scratch_types=[
            pltpu.VMEM((NGRP * MAXN,), jnp.int32),
            pltpu.VMEM((_ROWS_PER_W // _GROW, _GROW), jnp.int32),
            pltpu.VMEM((_ROWS_PER_W // _GROW, _GROW), jnp.int32),
            pltpu.VMEM((2, _GROW, C), jnp.float32),
            pltpu.SemaphoreType.DMA,
            pltpu.SemaphoreType.DMA,
        ],
        compiler_params=pltpu.CompilerParams(needs_layout_passes=False),
    )


def _sc_gather_body(t_hbm, tidx_hbm, feat_hbm, vf_hbm,
                    t_v, tidx_v, vidx_v, rows_v, gsem, osem):
    wid = lax.axis_index("s") * 2 + lax.axis_index("c")
    nrow = _ROWS_PER_W // _GROW
    pltpu.sync_copy(t_hbm, t_v)
    pltpu.sync_copy(tidx_hbm.at[pl.ds(wid * nrow, nrow)], tidx_v)
    for j in range(nrow):
        for l in range(_GROW // 16):
            tv = tidx_v[j, pl.ds(l * 16, 16)]
            vidx_v[j, pl.ds(l * 16, 16)] = plsc.load_gather(t_v, [tv])
    base = wid * _ROWS_PER_W
    pend = pltpu.async_copy(feat_hbm.at[vidx_v.at[0]], rows_v.at[0], gsem)
    for j in range(nrow):
        pend.wait()
        if j + 1 < nrow:
            pend = pltpu.async_copy(feat_hbm.at[vidx_v.at[j + 1]],
                                    rows_v.at[(j + 1) % 2], gsem)
        pltpu.async_copy(
            rows_v.at[j % 2], vf_hbm.at[pl.ds(base + j * _GROW, _GROW)],
            osem).wait()


_QT = 64


def _attn_body(qf_ref, vf_ref, cellt_ref, qx_ref, valid_ref,
               w1_ref, b1_ref, w2_ref, b2_ref, wq_ref, bq_ref,
               wk_ref, bk_ref, wv_ref, bv_ref, wo_ref, bo_ref, out_ref):
    cellt = cellt_ref[0]
    qx0 = qx_ref[0][:, 0:1]
    qx1 = qx_ref[0][:, 1:2]
    qx2 = qx_ref[0][:, 2:3]
    rx = (cellt % 4).astype(jnp.float32) - 1.5 - qx0
    ry = ((cellt // 4) % 4).astype(jnp.float32) - 1.5 - qx1
    rz = (cellt // 16).astype(jnp.float32) - 1.5 - qx2
    dd = jnp.sqrt(rx * rx + ry * ry + rz * rz + 1e-12)
    h3 = (rx[:, :, None] * w1_ref[0, :][None, None, :]
          + ry[:, :, None] * w1_ref[1, :][None, None, :]
          + rz[:, :, None] * w1_ref[2, :][None, None, :]
          + dd[:, :, None] * w1_ref[3, :][None, None, :]
          + b1_ref[...][None, :, :])
    h = h3.reshape(_QT * MAXN, C)
    pe = jnp.dot(jnp.maximum(h, 0.0), w2_ref[...],
                 preferred_element_type=jnp.float32, precision=lax.Precision.HIGHEST) + b2_ref[...]
    val = vf_ref[0] + pe
    q = jnp.dot(qf_ref[0], wq_ref[...],
                preferred_element_type=jnp.float32, precision=lax.Precision.HIGHEST) + bq_ref[...]
    kk = jnp.dot(val, wk_ref[...],
                 preferred_element_type=jnp.float32, precision=lax.Precision.HIGHEST) + bk_ref[...]
    vv = jnp.dot(val, wv_ref[...],
                 preferred_element_type=jnp.float32, precision=lax.Precision.HIGHEST) + bv_ref[...]
    kk3 = kk.reshape(_QT, MAXN, C)
    vv3 = vv.reshape(_QT, MAXN, C)
    logits = jnp.sum(q[:, None, :] * kk3, axis=2) * SCALE
    vmask = valid_ref[0] > 0.0
    logits = jnp.where(vmask, logits, -1e30)
    m = jnp.max(logits, axis=1, keepdims=True)
    e = jnp.exp(logits - m)
    attn = e / jnp.sum(e, axis=1, keepdims=True)
    agg = jnp.sum(attn[:, :, None] * vv3, axis=1)
    ob = jnp.dot(agg, wo_ref[...],
                 preferred_element_type=jnp.float32, precision=lax.Precision.HIGHEST) + bo_ref[...]
    anyv = jnp.max(valid_ref[0], axis=1, keepdims=True) > 0.0
    out_ref[0] = jnp.where(anyv, ob, 0.0)


def _attention(qf3, vf3, cellt3, qx3, valid3, w1, b1, w2, b2,
               wq, bq, wk, bk, wv, bv, wo, bo):
    nblk = (B * K) // _QT
    wspec = pl.BlockSpec((C, C), lambda i: (0, 0))
    bspec = pl.BlockSpec((1, C), lambda i: (0, 0))
    return pl.pallas_call(
        _attn_body,
        grid=(nblk,),
        in_specs=[
            pl.BlockSpec((1, _QT, C), lambda i: (i, 0, 0)),
            pl.BlockSpec((1, _QT * MAXN, C), lambda i: (i, 0, 0)),
            pl.BlockSpec((1, _QT, MAXN), lambda i: (i, 0, 0)),
            pl.BlockSpec((1, _QT, 3), lambda i: (i, 0, 0)),
            pl.BlockSpec((1, _QT, MAXN), lambda i: (i, 0, 0)),
            pl.BlockSpec((4, C), lambda i: (0, 0)), bspec,
            wspec, bspec, wspec, bspec, wspec, bspec, wspec, bspec,
            wspec, bspec,
        ],
        out_specs=pl.BlockSpec((1, _QT, C), lambda i: (i, 0, 0)),
        out_shape=jax.ShapeDtypeStruct((nblk, _QT, C), jnp.float32),
    )(qf3, vf3, cellt3, qx3, valid3, w1, b1, w2, b2, wq, bq, wk, bk, wv, bv,
      wo, bo)


def kernel(q_feat, q_xyz, sp_feat, sp_indices_full, W_pos1, b_pos1, W_pos2,
           b_pos2, Wq, bq, Wk, bk, Wv, bv, Wo, bo):
    qx_flat = q_xyz.reshape(B * K, 3)
    codes = _compute_codes(sp_indices_full)
    dist_s, cell_s = _rank_cells(qx_flat)
    t_tab, counts = _get_sc_count()(codes)
    nblk = B * K // _QB
    tidx, cellt, valid = _slot_walk(
        dist_s.reshape(nblk, _QB, NCELL), cell_s.reshape(nblk, _QB, NCELL),
        counts.reshape(B, 1, NCELL), t_tab.reshape(B, NCELL, MAXN))
    vf = _get_sc_gather()(t_tab, tidx.reshape(-1, _GROW), sp_feat)
    out = _attention(
        q_feat.reshape((B * K) // _QT, _QT, C),
        vf.reshape((B * K) // _QT, _QT * MAXN, C),
        cellt.reshape((B * K) // _QT, _QT, MAXN),
        qx_flat.reshape((B * K) // _QT, _QT, 3),
        valid.reshape((B * K) // _QT, _QT, MAXN),
        W_pos1, b_pos1.reshape(1, C), W_pos2, b_pos2.reshape(1, C),
        Wq, bq.reshape(1, C), Wk, bk.reshape(1, C),
        Wv, bv.reshape(1, C), Wo, bo.reshape(1, C))
    return out.reshape(B, K, C)


# speedup vs baseline: 1.1306x; 1.1306x over previous
import functools

import jax
import jax.numpy as jnp
from jax import lax
from jax.experimental import pallas as pl
from jax.experimental.pallas import tpu as pltpu
from jax.experimental.pallas import tpu_sc as plsc

B, K, C = 4, 256, 128
L = 65536
RADIUS = 1.5
MAXN = 64
NCELL = 64
NGRP = B * NCELL
SCALE = 1.0 / (C ** 0.5)


_CODES_BLK = 8192


def _codes_body(spif_ref, codes_ref):
    blk = spif_ref[...]
    codes_ref[...] = (blk[:, 0] * 64 + blk[:, 1] * 16 + blk[:, 2] * 4
                      + blk[:, 3])


def _compute_codes(spif):
    return pl.pallas_call(
        _codes_body,
        grid=(L // _CODES_BLK,),
        in_specs=[pl.BlockSpec((_CODES_BLK, 4), lambda i: (i, 0))],
        out_specs=pl.BlockSpec((_CODES_BLK,), lambda i: (i,)),
        out_shape=jax.ShapeDtypeStruct((L,), jnp.int32),
    )(spif)


def _rank_body(qx_ref, dist_s_ref, cell_s_ref):
    qx = qx_ref[...]
    ci = lax.broadcasted_iota(jnp.int32, (1, NCELL), 1)
    px = (ci % 4).astype(jnp.float32) - 1.5
    py = ((ci // 4) % 4).astype(jnp.float32) - 1.5
    pz = (ci // 16).astype(jnp.float32) - 1.5
    qx0, qx1, qx2 = qx[:, 0:1], qx[:, 1:2], qx[:, 2:3]
    qb0 = qx0.astype(jnp.bfloat16).astype(jnp.float32)
    qb1 = qx1.astype(jnp.bfloat16).astype(jnp.float32)
    qb2 = qx2.astype(jnp.bfloat16).astype(jnp.float32)
    qs = (qx0 * qx0 + qx2 * qx2) + qx1 * qx1
    ps = (px * px + pz * pz) + py * py
    dot = qb0 * px + qb1 * py + qb2 * pz
    d2 = (qs + ps) - 2.0 * dot
    dist = jnp.sqrt(jnp.maximum(d2, 0.0))

    ci64 = lax.broadcasted_iota(jnp.int32, (B * K, NCELL), 1)
    rank = jnp.zeros((B * K, NCELL), jnp.int32)
    for cp in range(NCELL):
        dcp = dist[:, cp:cp + 1]
        rank = rank + (dcp < dist).astype(jnp.int32) \
            + jnp.where((dcp == dist) & (cp < ci64), 1, 0)
    dist_s = jnp.zeros_like(dist)
    cell_s = jnp.zeros((B * K, NCELL), jnp.int32)
    for c0 in range(NCELL):
        m = rank[:, c0:c0 + 1] == ci64
        dist_s = jnp.where(m, dist[:, c0:c0 + 1], dist_s)
        cell_s = jnp.where(m, c0, cell_s)
    dist_s_ref[...] = dist_s
    cell_s_ref[...] = cell_s


def _rank_cells(qx_flat):
    return pl.pallas_call(
        _rank_body,
        out_shape=(jax.ShapeDtypeStruct((B * K, NCELL), jnp.float32),
                   jax.ShapeDtypeStruct((B * K, NCELL), jnp.int32)),
    )(qx_flat)


_SC_CHUNK = 4096


@functools.cache
def _get_sc_count():
    mesh = plsc.VectorSubcoreMesh(core_axis_name="c", subcore_axis_name="s")
    return pl.kernel(
        _sc_count_body,
        out_type=(jax.ShapeDtypeStruct((NGRP * MAXN,), jnp.int32),
                  jax.ShapeDtypeStruct((NGRP,), jnp.int32)),
        mesh=mesh,
        scratch_types=[
            pltpu.VMEM((_SC_CHUNK,), jnp.int32),
            pltpu.VMEM((NGRP * MAXN,), jnp.int32),
            pltpu.VMEM((NGRP,), jnp.int32),
        ],
        compiler_params=pltpu.CompilerParams(needs_layout_passes=False),
    )


def _sc_count_body(codes_hbm, t_hbm, counts_hbm, codes_v, t_v, cnt_v):
    wid = lax.axis_index("s") * 2 + lax.axis_index("c")

    @pl.when(wid == 0)
    def _():
        zeros16 = jnp.zeros((16,), jnp.int32)
        for i in range(NGRP // 16):
            cnt_v[pl.ds(i * 16, 16)] = zeros16

        def zero_t(i, _):
            t_v[pl.ds(i * 16, 16)] = zeros16
            return _
        lax.fori_loop(0, NGRP * MAXN // 16, zero_t, 0)

        iota16 = lax.iota(jnp.int32, 16)
        zc, _ = plsc.scan_count(zeros16)
        basev = zc - iota16

        def chunk_body(ch, _):
            pltpu.sync_copy(codes_hbm.at[pl.ds(ch * _SC_CHUNK, _SC_CHUNK)],
                            codes_v)

            def step(j, _):
                g = codes_v[pl.ds(j * 16, 16)]
                cn = plsc.load_gather(cnt_v, [g])
                dup, lastm = plsc.scan_count(g)
                pos = cn + (dup - basev)
                wm = pos < MAXN
                idxv = ch * _SC_CHUNK + j * 16 + iota16
                plsc.store_scatter(t_v, [g * MAXN + pos], idxv, mask=wm)
                plsc.store_scatter(cnt_v, [g], pos + 1, mask=lastm)
                return _
            lax.fori_loop(0, _SC_CHUNK // 16, step, 0)
            return _
        lax.fori_loop(0, L // _SC_CHUNK, chunk_body, 0)

        pltpu.sync_copy(t_v, t_hbm)
        pltpu.sync_copy(cnt_v, counts_hbm)


_QB = 64
_JMAX = 8


def _walk(cnt, cell_s, t64):
    q = cnt.shape[0]
    s_inc = cnt
    for sh in (1, 2, 4, 8, 16, 32):
        s_inc = s_inc + jnp.concatenate(
            [jnp.zeros((q, sh), jnp.int32), s_inc[:, :NCELL - sh]], axis=1)
    s_exc = s_inc - cnt
    r_t = jnp.zeros((q, NCELL), jnp.int32)
    for r0 in range(NCELL):
        r_t = r_t + (s_inc[:, r0:r0 + 1] <= t64).astype(jnp.int32)
    rc = jnp.minimum(r_t, NCELL - 1)
    cellt = jnp.zeros((q, NCELL), jnp.int32)
    sexg = jnp.zeros((q, NCELL), jnp.int32)
    for r0 in range(NCELL):
        msk = rc == r0
        cellt = jnp.where(msk, cell_s[:, r0:r0 + 1], cellt)
        sexg = jnp.where(msk, s_exc[:, r0:r0 + 1], sexg)
    o_i = jnp.clip(t64 - sexg, 0, MAXN - 1)
    return s_inc, s_exc, r_t, rc, cellt, o_i


def _slots_body(dist_s_ref, cell_s_ref, counts_ref, t_ref,
                tidx_ref, cellt_ref, valid_ref):
    b = pl.program_id(0) // (B * K // _QB // B)
    dist_s = dist_s_ref[0]
    cell_s = cell_s_ref[0]
    cnt_s = jnp.zeros((_QB, NCELL), jnp.int32)
    for c0 in range(NCELL):
        cnt_s = jnp.where(cell_s == c0, counts_ref[0, 0, c0], cnt_s)
    t64 = lax.broadcasted_iota(jnp.int32, (_QB, NCELL), 1)
    s_inc, s_exc, r_t, rc, cellt, o_i = _walk(cnt_s, cell_s, t64)
    distg = jnp.zeros((_QB, NCELL), jnp.float32)
    for r0 in range(NCELL):
        distg = jnp.where(rc == r0, dist_s[:, r0:r0 + 1], distg)
    ok = r_t < NCELL
    valid = ok & (distg <= RADIUS)
    tidx_ref[0] = (b * NCELL + cellt) * MAXN + o_i
    cellt_ref[0] = cellt
    valid_ref[0] = valid.astype(jnp.float32)

    neq = jnp.concatenate(
        [jnp.zeros((_QB, 1), jnp.int32),
         (dist_s[:, 1:] != dist_s[:, :-1]).astype(jnp.int32)], axis=1)
    runid = neq
    for sh in (1, 2, 4, 8, 16, 32):
        runid = runid + jnp.concatenate(
            [jnp.zeros((_QB, sh), jnp.int32), runid[:, :NCELL - sh]], axis=1)
    r_star = r_t[:, NCELL - 1:NCELL]
    rc_star = jnp.minimum(r_star, NCELL - 1)
    run_star = jnp.zeros((_QB, 1), jnp.int32)
    r_lo = jnp.zeros((_QB, 1), jnp.int32)
    r_hi = jnp.zeros((_QB, 1), jnp.int32)
    rexc = jnp.zeros((_QB, 1), jnp.int32)
    rinc = jnp.zeros((_QB, 1), jnp.int32)
    for r0 in range(NCELL):
        run_star = jnp.where(rc_star == r0, runid[:, r0:r0 + 1], run_star)
    for r0 in range(NCELL):
        r_lo = r_lo + (runid[:, r0:r0 + 1] < run_star).astype(jnp.int32)
        r_hi = r_hi + (runid[:, r0:r0 + 1] <= run_star).astype(jnp.int32)
    r_hi = r_hi - 1
    for r0 in range(NCELL):
        rexc = jnp.where(r_lo == r0, s_exc[:, r0:r0 + 1], rexc)
        rinc = jnp.where(r_hi == r0, s_inc[:, r0:r0 + 1], rinc)
    jrun = r_hi - r_lo + 1
    mneed = MAXN - rexc
    dist_star = distg[:, NCELL - 1:NCELL]
    flag = ((r_star < NCELL) & (jrun >= 2) & (jrun <= _JMAX)
            & (rinc > MAXN) & (dist_star <= RADIUS))
    anyflag = jnp.max(flag.astype(jnp.int32)) > 0

    @pl.when(anyflag)
    def _fix():
        cap_s = jnp.minimum(cnt_s, MAXN)
        iota_r = lax.broadcasted_iota(jnp.int32, (_QB, NCELL), 1)
        cellrun = jnp.zeros((_QB, _JMAX), jnp.int32)
        caprun = jnp.zeros((_QB, _JMAX), jnp.int32)
        iota_jk = lax.broadcasted_iota(jnp.int32, (_QB, _JMAX), 1)
        for k in range(_JMAX):
            msk = (iota_r == r_lo + k).astype(jnp.int32)
            cellrun = jnp.where(iota_jk == k,
                                jnp.sum(msk * cell_s, axis=1, keepdims=True),
                                cellrun)
            caprun = jnp.where(iota_jk == k,
                               jnp.sum(msk * cap_s, axis=1, keepdims=True),
                               caprun)
        ncol = _JMAX * MAXN
        col = lax.broadcasted_iota(jnp.int32, (_QB, ncol), 1)
        k_of = col // MAXN
        t_of = col % MAXN
        cellrun_w = jnp.zeros((_QB, ncol), jnp.int32)
        caprun_w = jnp.zeros((_QB, ncol), jnp.int32)
        for k in range(_JMAX):
            km = k_of == k
            cellrun_w = jnp.where(km, cellrun[:, k:k + 1], cellrun_w)
            caprun_w = jnp.where(km, caprun[:, k:k + 1], caprun_w)
        t3 = jnp.zeros((_QB, ncol), jnp.float32)
        for c0 in range(NCELL):
            trow = t_ref[0, c0].astype(jnp.float32)
            trow_w = jnp.concatenate([trow] * _JMAX, axis=0)[None, :]
            t3 = t3 + (cellrun_w == c0).astype(jnp.float32) * trow_w
        emask = (k_of < jrun) & (t_of < caprun_w)
        mf = mneed.astype(jnp.float32)
        v = jnp.zeros((_QB, 1), jnp.float32)
        for bit in range(15, -1, -1):
            cand = v + float(1 << bit)
            cnt_lt = jnp.sum(jnp.where(emask & (t3 < cand), 1.0, 0.0),
                             axis=1, keepdims=True)
            v = jnp.where(cnt_lt < mf, cand, v)
        sel = jnp.where(emask & (t3 < v + 1.0), 1.0, 0.0)
        cnt2 = cnt_s
        for k in range(_JMAX):
            pk = jnp.sum(jnp.where(k_of == k, sel, 0.0), axis=1,
                         keepdims=True).astype(jnp.int32)
            mskr = (iota_r == r_lo + k) & flag & (k < jrun)
            cnt2 = jnp.where(mskr, pk, cnt2)
        _, _, _, _, cellt2, o2 = _walk(cnt2, cell_s, t64)
        flagb = flag
        tidx_ref[0] = jnp.where(flagb, (b * NCELL + cellt2) * MAXN + o2,
                                tidx_ref[0])
        cellt_ref[0] = jnp.where(flagb, cellt2, cellt_ref[0])


def _slot_walk(dist_s3, cell_s3, counts4, t4):
    nblk = B * K // _QB
    per_b = nblk // B
    return pl.pallas_call(
        _slots_body,
        grid=(nblk,),
        in_specs=[
            pl.BlockSpec((1, _QB, NCELL), lambda i: (i, 0, 0)),
            pl.BlockSpec((1, _QB, NCELL), lambda i: (i, 0, 0)),
            pl.BlockSpec((1, 1, NCELL), lambda i: (i // per_b, 0, 0)),
            pl.BlockSpec((1, NCELL, MAXN), lambda i: (i // per_b, 0, 0)),
        ],
        out_specs=(
            pl.BlockSpec((1, _QB, MAXN), lambda i: (i, 0, 0)),
            pl.BlockSpec((1, _QB, MAXN), lambda i: (i, 0, 0)),
            pl.BlockSpec((1, _QB, MAXN), lambda i: (i, 0, 0)),
        ),
        out_shape=(
            jax.ShapeDtypeStruct((nblk, _QB, MAXN), jnp.int32),
            jax.ShapeDtypeStruct((nblk, _QB, MAXN), jnp.int32),
            jax.ShapeDtypeStruct((nblk, _QB, MAXN), jnp.float32),
        ),
    )(dist_s3, cell_s3, counts4, t4)


_ROWS_PER_W = (B * K * MAXN) // 32
_GROW = 128


@functools.cache
def _get_sc_gather():
    mesh = plsc.VectorSubcoreMesh(core_axis_name="c", subcore_axis_name="s")
    return pl.kernel(
        _sc_gather_body,
        out_type=jax.ShapeDtypeStruct((B * K * MAXN, C), jnp.float32),
        mesh=mesh,
        scratch_types=[
            pltpu.VMEM((NGRP * MAXN,), jnp.int32),
            pltpu.VMEM((_ROWS_PER_W // _GROW, _GROW), jnp.int32),
            pltpu.VMEM((_ROWS_PER_W // _GROW, _GROW), jnp.int32),
            pltpu.VMEM((2, _GROW, C), jnp.float32),
            pltpu.SemaphoreType.DMA,
            pltpu.SemaphoreType.DMA,
        ],
        compiler_params=pltpu.CompilerParams(needs_layout_passes=False),
    )


def _sc_gather_body(t_hbm, tidx_hbm, feat_hbm, vf_hbm,
                    t_v, tidx_v, vidx_v, rows_v, gsem, osem):
    wid = lax.axis_index("s") * 2 + lax.axis_index("c")
    nrow = _ROWS_PER_W // _GROW
    pltpu.sync_copy(t_hbm, t_v)
    pltpu.sync_copy(tidx_hbm.at[pl.ds(wid * nrow, nrow)], tidx_v)
    for j in range(nrow):
        for l in range(_GROW // 16):
            tv = tidx_v[j, pl.ds(l * 16, 16)]
            vidx_v[j, pl.ds(l * 16, 16)] = plsc.load_gather(t_v, [tv])
    base = wid * _ROWS_PER_W
    pend = pltpu.async_copy(feat_hbm.at[vidx_v.at[0]], rows_v.at[0], gsem)
    for j in range(nrow):
        pend.wait()
        if j + 1 < nrow:
            pend = pltpu.async_copy(feat_hbm.at[vidx_v.at[j + 1]],
                                    rows_v.at[(j + 1) % 2], gsem)
        pltpu.async_copy(
            rows_v.at[j % 2], vf_hbm.at[pl.ds(base + j * _GROW, _GROW)],
            osem).wait()


_QT = 64


def _attn_body(qf_ref, vf_ref, cellt_ref, qx_ref, valid_ref,
               w1_ref, b1_ref, w2_ref, b2_ref, wq_ref, bq_ref,
               wk_ref, bk_ref, wv_ref, bv_ref, wo_ref, bo_ref, out_ref):
    cellt = cellt_ref[0]
    qx0 = qx_ref[0][:, 0:1]
    qx1 = qx_ref[0][:, 1:2]
    qx2 = qx_ref[0][:, 2:3]
    rx = (cellt % 4).astype(jnp.float32) - 1.5 - qx0
    ry = ((cellt // 4) % 4).astype(jnp.float32) - 1.5 - qx1
    rz = (cellt // 16).astype(jnp.float32) - 1.5 - qx2
    dd = jnp.sqrt(rx * rx + ry * ry + rz * rz + 1e-12)
    h3 = (rx[:, :, None] * w1_ref[0, :][None, None, :]
          + ry[:, :, None] * w1_ref[1, :][None, None, :]
          + rz[:, :, None] * w1_ref[2, :][None, None, :]
          + dd[:, :, None] * w1_ref[3, :][None, None, :]
          + b1_ref[...][None, :, :])
    h = h3.reshape(_QT * MAXN, C)
    pe = jnp.dot(jnp.maximum(h, 0.0), w2_ref[...],
                 preferred_element_type=jnp.float32) + b2_ref[...]
    val = vf_ref[0] + pe
    q = jnp.dot(qf_ref[0], wq_ref[...],
                preferred_element_type=jnp.float32) + bq_ref[...]
    kk = jnp.dot(val, wk_ref[...],
                 preferred_element_type=jnp.float32) + bk_ref[...]
    vv = jnp.dot(val, wv_ref[...],
                 preferred_element_type=jnp.float32) + bv_ref[...]
    kk3 = kk.reshape(_QT, MAXN, C)
    vv3 = vv.reshape(_QT, MAXN, C)
    logits = jnp.sum(q[:, None, :] * kk3, axis=2) * SCALE
    vmask = valid_ref[0] > 0.0
    logits = jnp.where(vmask, logits, -1e30)
    m = jnp.max(logits, axis=1, keepdims=True)
    e = jnp.exp(logits - m)
    attn = e / jnp.sum(e, axis=1, keepdims=True)
    agg = jnp.sum(attn[:, :, None] * vv3, axis=1)
    ob = jnp.dot(agg, wo_ref[...],
                 preferred_element_type=jnp.float32) + bo_ref[...]
    anyv = jnp.max(valid_ref[0], axis=1, keepdims=True) > 0.0
    out_ref[0] = jnp.where(anyv, ob, 0.0)


def _attention(qf3, vf3, cellt3, qx3, valid3, w1, b1, w2, b2,
               wq, bq, wk, bk, wv, bv, wo, bo):
    nblk = (B * K) // _QT
    wspec = pl.BlockSpec((C, C), lambda i: (0, 0))
    bspec = pl.BlockSpec((1, C), lambda i: (0, 0))
    return pl.pallas_call(
        _attn_body,
        grid=(nblk,),
        in_specs=[
            pl.BlockSpec((1, _QT, C), lambda i: (i, 0, 0)),
            pl.BlockSpec((1, _QT * MAXN, C), lambda i: (i, 0, 0)),
            pl.BlockSpec((1, _QT, MAXN), lambda i: (i, 0, 0)),
            pl.BlockSpec((1, _QT, 3), lambda i: (i, 0, 0)),
            pl.BlockSpec((1, _QT, MAXN), lambda i: (i, 0, 0)),
            pl.BlockSpec((4, C), lambda i: (0, 0)), bspec,
            wspec, bspec, wspec, bspec, wspec, bspec, wspec, bspec,
            wspec, bspec,
        ],
        out_specs=pl.BlockSpec((1, _QT, C), lambda i: (i, 0, 0)),
        out_shape=jax.ShapeDtypeStruct((nblk, _QT, C), jnp.float32),
    )(qf3, vf3, cellt3, qx3, valid3, w1, b1, w2, b2, wq, bq, wk, bk, wv, bv,
      wo, bo)


def kernel(q_feat, q_xyz, sp_feat, sp_indices_full, W_pos1, b_pos1, W_pos2,
           b_pos2, Wq, bq, Wk, bk, Wv, bv, Wo, bo):
    qx_flat = q_xyz.reshape(B * K, 3)
    codes = _compute_codes(sp_indices_full)
    dist_s, cell_s = _rank_cells(qx_flat)
    t_tab, counts = _get_sc_count()(codes)
    nblk = B * K // _QB
    tidx, cellt, valid = _slot_walk(
        dist_s.reshape(nblk, _QB, NCELL), cell_s.reshape(nblk, _QB, NCELL),
        counts.reshape(B, 1, NCELL), t_tab.reshape(B, NCELL, MAXN))
    vf = _get_sc_gather()(t_tab, tidx.reshape(-1, _GROW), sp_feat)
    out = _attention(
        q_feat.reshape((B * K) // _QT, _QT, C),
        vf.reshape((B * K) // _QT, _QT * MAXN, C),
        cellt.reshape((B * K) // _QT, _QT, MAXN),
        qx_flat.reshape((B * K) // _QT, _QT, 3),
        valid.reshape((B * K) // _QT, _QT, MAXN),
        W_pos1, b_pos1.reshape(1, C), W_pos2, b_pos2.reshape(1, C),
        Wq, bq.reshape(1, C), Wk, bk.reshape(1, C),
        Wv, bv.reshape(1, C), Wo, bo.reshape(1, C))
    return out.reshape(B, K, C)
